# Initial kernel scaffold; baseline (speedup 1.0000x reference)
import jax
import jax.numpy as jnp
from jax.experimental import pallas as pl


def kernel(genre_ids_batch, embedding_weight):
    raise NotImplementedError("write your pallas kernel here")


# speedup vs baseline: 9.0303x; 9.0303x over previous
import functools

import jax
import jax.numpy as jnp
from jax import lax
from jax.experimental import pallas as pl
from jax.experimental.pallas import tpu as pltpu
from jax.experimental.pallas import tpu_sc as plsc

_B = 16384
_L = 200
_D = 32
_V = 1001
_NC = 2
_NS = 16
_LANES = 16
_NW = _NC * _NS
_RPB = _LANES
_NBLK = _B // _RPB
_BPW = _NBLK // _NW


def _sc_body(ids_hbm, tab_hbm, out_hbm, tab_v, ids_v, out_v):
    wid = lax.axis_index("s") * _NC + lax.axis_index("c")
    pltpu.sync_copy(tab_hbm, tab_v)
    row_iota = lax.iota(jnp.int32, _LANES)

    def block_body(i, carry):
        blk = wid * _BPW + i
        pltpu.sync_copy(ids_hbm.at[blk], ids_v)

        init = (jnp.zeros((_LANES,), jnp.int32),) + tuple(
            jnp.zeros((_LANES,), jnp.float32) for _ in range(_D)
        )

        def l_body(l, c):
            cnt = c[0]
            accs = c[1:]
            ids16 = ids_v[l]
            cnt = cnt + (ids16 != 0).astype(jnp.int32)
            base = ids16 * _D
            new = []
            for d in range(_D):
                g = plsc.load_gather(tab_v, [base + d])
                new.append(accs[d] + g)
            return (cnt,) + tuple(new)

        res = lax.fori_loop(0, _L, l_body, init)
        cnt = res[0]
        inv = 1.0 / jnp.maximum(cnt.astype(jnp.float32), 1.0)
        for d in range(_D):
            plsc.store_scatter(
                out_v, [row_iota, jnp.full((_LANES,), d, jnp.int32)],
                res[1 + d] * inv)
        pltpu.sync_copy(out_v, out_hbm.at[pl.ds(blk * _RPB, _RPB)])
        return carry

    lax.fori_loop(0, _BPW, block_body, 0)


@jax.jit
def kernel(genre_ids_batch, embedding_weight):
    ids_t = genre_ids_batch.reshape(_NBLK, _RPB, _L).transpose(0, 2, 1)
    tab_flat = embedding_weight.reshape(_V * _D)
    call = pl.kernel(
        _sc_body,
        out_type=jax.ShapeDtypeStruct((_B, _D), jnp.float32),
        mesh=plsc.VectorSubcoreMesh(
            core_axis_name="c", subcore_axis_name="s",
            num_cores=_NC, num_subcores=_NS),
        scratch_types=[
            pltpu.VMEM((_V * _D,), jnp.float32),
            pltpu.VMEM((_L, _RPB), jnp.int32),
            pltpu.VMEM((_RPB, _D), jnp.float32),
        ],
        compiler_params=pltpu.CompilerParams(use_tc_tiling_on_sc=False, needs_layout_passes=False),
    )
    return call(ids_t, tab_flat)
